# baseline (speedup 1.0000x reference)
import jax
import jax.numpy as jnp
from jax.experimental import pallas as pl
from jax.experimental.pallas import tpu as pltpu

_BR = 512


def _body(xt_ref, adj_ref, w_ref, b_ref, keept_ref, out_ref,
          deg_ref, adj_bf):
    t = pl.program_id(0)
    nt = pl.num_programs(0) - 1

    @pl.when(t == 0)
    def _init_deg():
        deg_ref[...] = jnp.ones_like(deg_ref)

    @pl.when(t < nt)
    def _deg_and_park():
        adj = adj_ref[...]
        adj_bf[pl.ds(t * _BR, _BR), :] = adj.astype(jnp.bfloat16)
        ones_row = jnp.ones((1, _BR), dtype=jnp.float32)
        deg_ref[...] += jax.lax.dot_general(
            ones_row, adj, (((1,), (0,)), ((), ())),
            preferred_element_type=jnp.float32,
        )

    @pl.when(t == nt)
    def _compute():
        dinv = jax.lax.rsqrt(deg_ref[...])
        ytd = xt_ref[...] * keept_ref[...] * 2.0 * dinv
        zt = jax.lax.dot_general(
            ytd.astype(jnp.bfloat16), adj_bf[...],
            (((1,), (0,)), ((), ())),
            preferred_element_type=jnp.float32,
        )
        zt = (zt + ytd) * dinv
        ht = jax.lax.dot_general(
            w_ref[...], zt, (((1,), (0,)), ((), ())),
            preferred_element_type=jnp.float32,
            precision=jax.lax.Precision.HIGHEST,
        ) + b_ref[...]
        ht = ht - jnp.mean(ht, axis=1, keepdims=True)
        rnorm = jnp.sqrt(1e-6 + jnp.sum(ht * ht, axis=0, keepdims=True))
        out_ref[...] = jnp.maximum(ht / rnorm, 0.0)


def kernel(x, adj, W, b):
    n, f = x.shape
    nt = n // _BR
    keep = jax.random.bernoulli(
        jax.random.key(42), 0.5, x.shape).astype(jnp.float32)
    out_t = pl.pallas_call(
        _body,
        grid=(nt + 1,),
        in_specs=[
            pl.BlockSpec((f, n), lambda t: (0, 0)),
            pl.BlockSpec((_BR, n),
                         lambda t: (jnp.minimum(t, nt - 1), 0)),
            pl.BlockSpec((f, f), lambda t: (0, 0)),
            pl.BlockSpec((f, 1), lambda t: (0, 0)),
            pl.BlockSpec((f, n), lambda t: (0, 0)),
        ],
        out_specs=pl.BlockSpec((f, n), lambda t: (0, 0)),
        out_shape=jax.ShapeDtypeStruct((f, n), jnp.float32),
        scratch_shapes=[
            pltpu.VMEM((1, n), jnp.float32),
            pltpu.VMEM((n, n), jnp.bfloat16),
        ],
    )(x.T, adj, W, b.reshape(f, 1), keep.T)
    return (out_t.T, adj)


# speedup vs baseline: 1.0095x; 1.0095x over previous
import jax
import jax.numpy as jnp
from jax.experimental import pallas as pl


def _body(xt_ref, adj_ref, w_ref, b_ref, keept_ref, out_ref):
    adj = adj_ref[...]
    n = adj.shape[0]
    ones_row = jnp.ones((1, n), dtype=jnp.float32)
    deg = jax.lax.dot_general(
        ones_row, adj, (((1,), (0,)), ((), ())),
        preferred_element_type=jnp.float32,
    ) + 1.0
    dinv = jax.lax.rsqrt(deg)

    ytd = xt_ref[...] * keept_ref[...] * 2.0 * dinv
    zt = jax.lax.dot_general(
        ytd, adj, (((1,), (0,)), ((), ())),
        preferred_element_type=jnp.float32,
    )
    zt = (zt + ytd) * dinv

    ht = jax.lax.dot_general(
        w_ref[...], zt, (((1,), (0,)), ((), ())),
        preferred_element_type=jnp.float32,
        precision=jax.lax.Precision.HIGHEST,
    ) + b_ref[...]
    ht = ht - jnp.mean(ht, axis=1, keepdims=True)
    rnorm = jnp.sqrt(1e-6 + jnp.sum(ht * ht, axis=0, keepdims=True))
    out_ref[...] = jnp.maximum(ht / rnorm, 0.0)


def kernel(x, adj, W, b):
    n, f = x.shape
    keep = jax.random.bernoulli(
        jax.random.key(42), 0.5, x.shape).astype(jnp.float32)
    out_t = pl.pallas_call(
        _body,
        out_shape=jax.ShapeDtypeStruct((f, n), jnp.float32),
    )(x.T, adj, W, b.reshape(f, 1), keep.T)
    return (out_t.T, adj)
